# baseline (speedup 1.0000x reference)
import jax
import jax.numpy as jnp
from jax import lax
from jax.experimental import pallas as pl
from jax.experimental.pallas import tpu as pltpu
from jax.experimental.pallas import tpu_sc as plsc

F32 = jnp.float32
I32 = jnp.int32

NE = 50000
NU = 50000
NI = 30000
EDG = 800000
D = 64
H = 32
NREL = 32

NTILE = 16
E_PAD = 819200
EPT = E_PAD // NTILE

RP_ENT = 50176
Q_ENT = 3136
RP_ITM = 30208
Q_ITM = 1888


def _pad_idx(a, pad_vals, w):
    return jnp.concatenate([a.astype(I32), pad_vals]).reshape(E_PAD // w, w)


def _make_sc_kernel(r_pad, q, kch, grp, idxw, use_w, mean):
    mesh = plsc.VectorSubcoreMesh(core_axis_name="c", subcore_axis_name="s")
    out_type = [jax.ShapeDtypeStruct((r_pad, D), F32)]
    if mean:
        out_type.append(jax.ShapeDtypeStruct((r_pad,), F32))
        out_type.append(jax.ShapeDtypeStruct((r_pad,), F32))
    nsub = kch // idxw
    nchunk = EPT // kch
    gn = grp * nsub
    assert grp % 2 == 0 and nchunk % grp == 0
    ngroups = nchunk // grp
    scratch = [
        pltpu.VMEM((gn, idxw), I32),
        pltpu.VMEM((gn, idxw), I32),
        pltpu.VMEM((kch, H), F32),
        pltpu.VMEM((kch, H), F32),
        pltpu.VMEM_SHARED((r_pad, H), F32),
        pltpu.SemaphoreType.DMA,
        pltpu.SemaphoreType.DMA,
        pltpu.SemaphoreType.DMA,
    ]
    if use_w:
        scratch += [pltpu.VMEM((gn, idxw), I32),
                    pltpu.VMEM((kch, H // 2), I32),
                    pltpu.VMEM((kch, H // 2), I32)]
    if mean:
        scratch += [pltpu.VMEM_SHARED((r_pad,), F32),
                    pltpu.VMEM((idxw,), F32)]

    def body(*refs):
        src_lo, src_hi, gidx, sidx = refs[0], refs[1], refs[2], refs[3]
        i = 4
        if use_w:
            tyidx, w_lo, w_hi = refs[i], refs[i + 1], refs[i + 2]
            i += 3
        zrows = refs[i]
        i += 1
        if mean:
            zcnt, ones_h = refs[i], refs[i + 1]
            i += 2
        out_rows = refs[i]
        i += 1
        if mean:
            out_cnt_a, out_cnt_b = refs[i], refs[i + 1]
            i += 2
        gi_v, si_v, rows0, rows1, acc, sem0, sem1, ssem = refs[i:i + 8]
        i += 8
        if use_w:
            ti_v, wrows0, wrows1 = refs[i], refs[i + 1], refs[i + 2]
            i += 3
        if mean:
            cnt, ones_v = refs[i], refs[i + 1]

        c = lax.axis_index("c")
        rows_b = (rows0, rows1)
        wrows_b = (wrows0, wrows1) if use_w else None
        sems = (sem0, sem1)

        def run(src, wtab, col0, cnt_out_ref):
            t = lax.axis_index("s")
            pltpu.sync_copy(zrows, acc.at[pl.ds(t * q, q)])
            if mean:
                @pl.when(t == 0)
                def _():
                    pltpu.sync_copy(zcnt, cnt)
                pltpu.sync_copy(ones_h, ones_v)
            plsc.subcore_barrier()

            def issue_g(ck, pb):
                for j in range(nsub):
                    r = ck * nsub + j
                    pltpu.async_copy(src.at[gi_v.at[r]],
                                     rows_b[pb].at[pl.ds(j * idxw, idxw)],
                                     sems[pb])
                    if use_w:
                        pltpu.async_copy(wtab.at[ti_v.at[r]],
                                         wrows_b[pb].at[pl.ds(j * idxw, idxw)],
                                         sems[pb])

            def wait_g(ck, pb):
                for j in range(nsub):
                    r = ck * nsub + j
                    pltpu.make_async_copy(
                        src.at[gi_v.at[r]],
                        rows_b[pb].at[pl.ds(j * idxw, idxw)], sems[pb]).wait()
                    if use_w:
                        pltpu.make_async_copy(
                            wtab.at[ti_v.at[r]],
                            wrows_b[pb].at[pl.ds(j * idxw, idxw)],
                            sems[pb]).wait()

            def do_chunk(ck, pb):
                wait_g(ck, pb)
                if use_w:
                    def mul16(m, cr):
                        for r8 in range(16):
                            rw = m * 16 + r8
                            raw = wrows_b[pb][rw, :]
                            wlo = lax.bitcast_convert_type(lax.shift_left(raw, 16), F32)
                            whi = lax.bitcast_convert_type(
                                lax.bitwise_and(raw, jnp.int32(-65536)), F32)
                            sl0 = (rw, pl.ds(0, 16))
                            sl1 = (rw, pl.ds(16, 16))
                            rows_b[pb][sl0] = rows_b[pb][sl0] * wlo
                            rows_b[pb][sl1] = rows_b[pb][sl1] * whi
                        return cr
                    lax.fori_loop(0, kch // 16, mul16, 0)
                sds = []
                for j in range(nsub):
                    r = ck * nsub + j
                    sds.append(pltpu.async_copy(
                        rows_b[pb].at[pl.ds(j * idxw, idxw)],
                        acc.at[si_v.at[r]], ssem, add=True))
                for dd in sds:
                    dd.wait()
                if mean:
                    @pl.when((ck % 2) == c)
                    def _():
                        cds = []
                        for j in range(nsub):
                            r = ck * nsub + j
                            cds.append(pltpu.async_copy(
                                ones_v, cnt.at[si_v.at[r]], ssem, add=True))
                        for dd in cds:
                            dd.wait()

            def group(g, carry):
                base = t * (nchunk * nsub) + g * gn
                pltpu.sync_copy(gidx.at[pl.ds(base, gn)], gi_v)
                pltpu.sync_copy(sidx.at[pl.ds(base, gn)], si_v)
                if use_w:
                    pltpu.sync_copy(tyidx.at[pl.ds(base, gn)], ti_v)
                    toff = t * NREL
                    for r in range(gn):
                        for g8 in range(idxw // 16):
                            sl = (r, pl.ds(g8 * 16, 16))
                            ti_v[sl] = ti_v[sl] + toff
                issue_g(0, 0)

                def pair(p, cr):
                    a = 2 * p
                    issue_g(a + 1, 1)
                    do_chunk(a, 0)

                    @pl.when(p < grp // 2 - 1)
                    def _():
                        issue_g(a + 2, 0)
                    do_chunk(a + 1, 1)
                    return cr

                lax.fori_loop(0, grp // 2, pair, 0)
                return carry

            lax.fori_loop(0, ngroups, group, 0)
            plsc.subcore_barrier()
            pltpu.sync_copy(acc.at[pl.ds(t * q, q)],
                            out_rows.at[pl.ds(t * q, q), pl.ds(col0, H)])
            if mean:
                @pl.when(t == 0)
                def _():
                    pltpu.sync_copy(cnt, cnt_out_ref)

        @pl.when(c == 0)
        def _():
            run(src_lo, w_lo if use_w else None, 0,
                out_cnt_a if mean else None)

        @pl.when(c == 1)
        def _():
            run(src_hi, w_hi if use_w else None, H,
                out_cnt_b if mean else None)

    return pl.kernel(body, out_type=tuple(out_type), mesh=mesh,
                     scratch_types=scratch,
                     compiler_params=pltpu.CompilerParams(
                         use_tc_tiling_on_sc=False))


def _fusion_call(kg_sum, kca, kcb, in_sum, ica, icb, w1, w2):
    B = 600
    grid = (NI // B,)

    def fbody(kl, kca_r, kcb_r, il, ica_r, icb_r, w1r, w2r, fus, fl, fh, kc, ic):
        kg = kl[...] / jnp.maximum(kca_r[...] + kcb_r[...], 1.0)
        it = il[...] / jnp.maximum(ica_r[...] + icb_r[...], 1.0)
        z = lax.dot_general(kg, w1r[...], (((1,), (1,)), ((), ())),
                            preferred_element_type=F32)
        z = z + lax.dot_general(it, w2r[...], (((1,), (1,)), ((), ())),
                                preferred_element_type=F32)
        g = jax.nn.sigmoid(z)
        f = g * kg + (1.0 - g) * it
        fus[...] = f
        fl[...] = f[:, :H]
        fh[...] = f[:, H:]
        kc[...] = kg
        ic[...] = it

    bs_h = pl.BlockSpec((B, H), lambda i: (i, 0))
    bs_d = pl.BlockSpec((B, D), lambda i: (i, 0))
    bs_c = pl.BlockSpec((B, 1), lambda i: (i, 0))
    bs_w = pl.BlockSpec((D, D), lambda i: (0, 0))
    return pl.pallas_call(
        fbody, grid=grid,
        in_specs=[bs_d, bs_c, bs_c, bs_d, bs_c, bs_c, bs_w, bs_w],
        out_specs=[bs_d, bs_h, bs_h, bs_d, bs_d],
        out_shape=[jax.ShapeDtypeStruct((NI, D), F32),
                   jax.ShapeDtypeStruct((NI, H), F32),
                   jax.ShapeDtypeStruct((NI, H), F32),
                   jax.ShapeDtypeStruct((NI, D), F32),
                   jax.ShapeDtypeStruct((NI, D), F32)],
    )(kg_sum, kca, kcb, in_sum, ica, icb, w1, w2)


def _att_div_call(kg_sum, kca, kcb):
    NA = NE - NI
    B = 400
    grid = (NA // B,)

    def abody(kl, kca_r, kcb_r, out):
        out[...] = kl[...] / jnp.maximum(kca_r[...] + kcb_r[...], 1.0)

    return pl.pallas_call(
        abody, grid=grid,
        in_specs=[pl.BlockSpec((B, D), lambda i: (i, 0)),
                  pl.BlockSpec((B, 1), lambda i: (i, 0)),
                  pl.BlockSpec((B, 1), lambda i: (i, 0))],
        out_specs=pl.BlockSpec((B, D), lambda i: (i, 0)),
        out_shape=jax.ShapeDtypeStruct((NA, D), F32),
    )(kg_sum, kca, kcb)


def kernel(entity_emb, user_emb, edge_index, edge_type, interact_mat, weight,
           W1, W2):
    head = edge_index[0]
    tail = edge_index[1]
    row = interact_mat[0]
    col = interact_mat[1]
    ent_lo = entity_emb[:, :H]
    ent_hi = entity_emb[:, H:]
    usr_lo = user_emb[:, :H]
    usr_hi = user_emb[:, H:]
    def _pack_w(wh):
        wb = wh.astype(jnp.bfloat16)
        lo16 = jax.lax.bitcast_convert_type(wb[:, :16],
                                            jnp.uint16).astype(jnp.uint32)
        hi16 = jax.lax.bitcast_convert_type(wb[:, 16:],
                                            jnp.uint16).astype(jnp.uint32)
        packed = ((hi16 << 16) | lo16).astype(I32)
        return jnp.tile(packed, (NTILE, 1))

    w_lo = _pack_w(weight[:, :H])
    w_hi = _pack_w(weight[:, H:])

    npad = E_PAD - EDG
    zpad = jnp.zeros((npad,), I32)
    trash_ent = (jnp.arange(npad, dtype=I32) % (RP_ENT - NE)) + NE
    trash_itm = (jnp.arange(npad, dtype=I32) % (RP_ITM - NI)) + NI

    head_p = _pad_idx(head, trash_ent, 128)
    tail_p = _pad_idx(tail, zpad, 128)
    type_p = _pad_idx(edge_type, zpad, 128)
    rowg_p = _pad_idx(row, zpad, 256)
    row_p = _pad_idx(row, trash_ent, 256)
    colg_p = _pad_idx(col, zpad, 256)
    col_p = _pad_idx(col, trash_itm, 256)

    ones128 = jnp.ones((128,), F32)
    ones256 = jnp.ones((256,), F32)
    z_ent_rows = jnp.zeros((Q_ENT, H), F32)
    z_ent_cnt = jnp.zeros((RP_ENT,), F32)
    z_itm_rows = jnp.zeros((Q_ITM, H), F32)
    z_itm_cnt = jnp.zeros((RP_ITM,), F32)

    kg_k = _make_sc_kernel(RP_ENT, Q_ENT, 128, 20, 128, True, True)
    kg_sum, kg_ca, kg_cb = kg_k(ent_lo, ent_hi, tail_p, head_p, type_p,
                                w_lo, w_hi, z_ent_rows, z_ent_cnt, ones128)

    int_k = _make_sc_kernel(RP_ITM, Q_ITM, 512, 20, 256, False, True)
    int_sum, int_ca, int_cb = int_k(usr_lo, usr_hi, rowg_p, col_p,
                                    z_itm_rows, z_itm_cnt, ones256)

    fus, fus_lo, fus_hi, kg_cat, int_cat = _fusion_call(
        kg_sum[:NI], kg_ca[:NI, None], kg_cb[:NI, None],
        int_sum[:NI], int_ca[:NI, None], int_cb[:NI, None], W1, W2)

    usr_k = _make_sc_kernel(RP_ENT, Q_ENT, 256, 20, 256, False, False)
    user_agg_p = usr_k(fus_lo, fus_hi, colg_p, row_p, z_ent_rows)
    if isinstance(user_agg_p, (tuple, list)):
        user_agg_p = user_agg_p[0]

    att = _att_div_call(kg_sum[NI:NE], kg_ca[NI:NE, None], kg_cb[NI:NE, None])
    final_entity = jnp.concatenate([fus, att], axis=0)
    return final_entity, user_agg_p[:NU], kg_cat, int_cat


# speedup vs baseline: 1.0166x; 1.0166x over previous
import jax
import jax.numpy as jnp
from jax import lax
from jax.experimental import pallas as pl
from jax.experimental.pallas import tpu as pltpu
from jax.experimental.pallas import tpu_sc as plsc

F32 = jnp.float32
I32 = jnp.int32

NE = 50000
NU = 50000
NI = 30000
EDG = 800000
D = 64
H = 32
NREL = 32

NTILE = 16
E_PAD = 819200
EPT = E_PAD // NTILE

RP_ENT = 50176
Q_ENT = 3136
RP_ITM = 30208
Q_ITM = 1888


def _pad_idx(a, pad_vals, w):
    return jnp.concatenate([a.astype(I32), pad_vals]).reshape(E_PAD // w, w)


def _make_sc_kernel(r_pad, q, kch, grp, idxw, use_w, mean):
    mesh = plsc.VectorSubcoreMesh(core_axis_name="c", subcore_axis_name="s")
    out_type = [jax.ShapeDtypeStruct((r_pad, D), F32)]
    if mean:
        out_type.append(jax.ShapeDtypeStruct((r_pad,), F32))
        out_type.append(jax.ShapeDtypeStruct((r_pad,), F32))
    nsub = kch // idxw
    nchunk = EPT // kch
    gn = grp * nsub
    assert grp % 2 == 0 and nchunk % grp == 0
    ngroups = nchunk // grp
    scratch = [
        pltpu.VMEM((gn, idxw), I32),
        pltpu.VMEM((gn, idxw), I32),
        pltpu.VMEM((kch, H), F32),
        pltpu.VMEM((kch, H), F32),
        pltpu.VMEM_SHARED((r_pad, H), F32),
        pltpu.SemaphoreType.DMA,
        pltpu.SemaphoreType.DMA,
        pltpu.SemaphoreType.DMA,
    ]
    if use_w:
        scratch += [pltpu.VMEM((gn, idxw), I32),
                    pltpu.VMEM((kch, H), F32),
                    pltpu.VMEM((kch, H), F32)]
    if mean:
        scratch += [pltpu.VMEM_SHARED((r_pad,), F32),
                    pltpu.VMEM((idxw,), F32)]

    def body(*refs):
        src_lo, src_hi, gidx, sidx = refs[0], refs[1], refs[2], refs[3]
        i = 4
        if use_w:
            tyidx, w_lo, w_hi = refs[i], refs[i + 1], refs[i + 2]
            i += 3
        zrows = refs[i]
        i += 1
        if mean:
            zcnt, ones_h = refs[i], refs[i + 1]
            i += 2
        out_rows = refs[i]
        i += 1
        if mean:
            out_cnt_a, out_cnt_b = refs[i], refs[i + 1]
            i += 2
        gi_v, si_v, rows0, rows1, acc, sem0, sem1, ssem = refs[i:i + 8]
        i += 8
        if use_w:
            ti_v, wrows0, wrows1 = refs[i], refs[i + 1], refs[i + 2]
            i += 3
        if mean:
            cnt, ones_v = refs[i], refs[i + 1]

        c = lax.axis_index("c")
        rows_b = (rows0, rows1)
        wrows_b = (wrows0, wrows1) if use_w else None
        sems = (sem0, sem1)

        def run(src, wtab, col0, cnt_out_ref):
            t = lax.axis_index("s")
            pltpu.sync_copy(zrows, acc.at[pl.ds(t * q, q)])
            if mean:
                @pl.when(t == 0)
                def _():
                    pltpu.sync_copy(zcnt, cnt)
                pltpu.sync_copy(ones_h, ones_v)
            plsc.subcore_barrier()

            def issue_g(ck, pb):
                for j in range(nsub):
                    r = ck * nsub + j
                    pltpu.async_copy(src.at[gi_v.at[r]],
                                     rows_b[pb].at[pl.ds(j * idxw, idxw)],
                                     sems[pb])
                    if use_w:
                        pltpu.async_copy(wtab.at[ti_v.at[r]],
                                         wrows_b[pb].at[pl.ds(j * idxw, idxw)],
                                         sems[pb])

            def wait_g(ck, pb):
                for j in range(nsub):
                    r = ck * nsub + j
                    pltpu.make_async_copy(
                        src.at[gi_v.at[r]],
                        rows_b[pb].at[pl.ds(j * idxw, idxw)], sems[pb]).wait()
                    if use_w:
                        pltpu.make_async_copy(
                            wtab.at[ti_v.at[r]],
                            wrows_b[pb].at[pl.ds(j * idxw, idxw)],
                            sems[pb]).wait()

            def do_chunk(ck, pb):
                wait_g(ck, pb)
                if use_w:
                    def mul16(m, cr):
                        for r8 in range(16):
                            rw = m * 16 + r8
                            for hh in range(2):
                                sl = (rw, pl.ds(hh * 16, 16))
                                rows_b[pb][sl] = rows_b[pb][sl] * wrows_b[pb][sl]
                        return cr
                    lax.fori_loop(0, kch // 16, mul16, 0)
                sds = []
                for j in range(nsub):
                    r = ck * nsub + j
                    sds.append(pltpu.async_copy(
                        rows_b[pb].at[pl.ds(j * idxw, idxw)],
                        acc.at[si_v.at[r]], ssem, add=True))
                for dd in sds:
                    dd.wait()
                if mean:
                    @pl.when((ck % 2) == c)
                    def _():
                        cds = []
                        for j in range(nsub):
                            r = ck * nsub + j
                            cds.append(pltpu.async_copy(
                                ones_v, cnt.at[si_v.at[r]], ssem, add=True))
                        for dd in cds:
                            dd.wait()

            def group(g, carry):
                base = t * (nchunk * nsub) + g * gn
                pltpu.sync_copy(gidx.at[pl.ds(base, gn)], gi_v)
                pltpu.sync_copy(sidx.at[pl.ds(base, gn)], si_v)
                if use_w:
                    pltpu.sync_copy(tyidx.at[pl.ds(base, gn)], ti_v)
                    toff = t * NREL
                    for r in range(gn):
                        for g8 in range(idxw // 16):
                            sl = (r, pl.ds(g8 * 16, 16))
                            ti_v[sl] = ti_v[sl] + toff
                issue_g(0, 0)

                def pair(p, cr):
                    a = 2 * p
                    issue_g(a + 1, 1)
                    do_chunk(a, 0)

                    @pl.when(p < grp // 2 - 1)
                    def _():
                        issue_g(a + 2, 0)
                    do_chunk(a + 1, 1)
                    return cr

                lax.fori_loop(0, grp // 2, pair, 0)
                return carry

            lax.fori_loop(0, ngroups, group, 0)
            plsc.subcore_barrier()
            pltpu.sync_copy(acc.at[pl.ds(t * q, q)],
                            out_rows.at[pl.ds(t * q, q), pl.ds(col0, H)])
            if mean:
                @pl.when(t == 0)
                def _():
                    pltpu.sync_copy(cnt, cnt_out_ref)

        @pl.when(c == 0)
        def _():
            run(src_lo, w_lo if use_w else None, 0,
                out_cnt_a if mean else None)

        @pl.when(c == 1)
        def _():
            run(src_hi, w_hi if use_w else None, H,
                out_cnt_b if mean else None)

    return pl.kernel(body, out_type=tuple(out_type), mesh=mesh,
                     scratch_types=scratch,
                     compiler_params=pltpu.CompilerParams(
                         use_tc_tiling_on_sc=False))


def _fusion_call(kg_sum, kca, kcb, in_sum, ica, icb, w1, w2):
    B = 600
    grid = (NI // B,)

    def fbody(kl, kca_r, kcb_r, il, ica_r, icb_r, w1r, w2r, fus, fl, fh, kc, ic):
        kg = kl[...] / jnp.maximum(kca_r[...] + kcb_r[...], 1.0)
        it = il[...] / jnp.maximum(ica_r[...] + icb_r[...], 1.0)
        z = lax.dot_general(kg, w1r[...], (((1,), (1,)), ((), ())),
                            preferred_element_type=F32)
        z = z + lax.dot_general(it, w2r[...], (((1,), (1,)), ((), ())),
                                preferred_element_type=F32)
        g = jax.nn.sigmoid(z)
        f = g * kg + (1.0 - g) * it
        fus[...] = f
        fl[...] = f[:, :H]
        fh[...] = f[:, H:]
        kc[...] = kg
        ic[...] = it

    bs_h = pl.BlockSpec((B, H), lambda i: (i, 0))
    bs_d = pl.BlockSpec((B, D), lambda i: (i, 0))
    bs_c = pl.BlockSpec((B, 1), lambda i: (i, 0))
    bs_w = pl.BlockSpec((D, D), lambda i: (0, 0))
    return pl.pallas_call(
        fbody, grid=grid,
        in_specs=[bs_d, bs_c, bs_c, bs_d, bs_c, bs_c, bs_w, bs_w],
        out_specs=[bs_d, bs_h, bs_h, bs_d, bs_d],
        out_shape=[jax.ShapeDtypeStruct((NI, D), F32),
                   jax.ShapeDtypeStruct((NI, H), F32),
                   jax.ShapeDtypeStruct((NI, H), F32),
                   jax.ShapeDtypeStruct((NI, D), F32),
                   jax.ShapeDtypeStruct((NI, D), F32)],
    )(kg_sum, kca, kcb, in_sum, ica, icb, w1, w2)


def _att_div_call(kg_sum, kca, kcb):
    NA = NE - NI
    B = 400
    grid = (NA // B,)

    def abody(kl, kca_r, kcb_r, out):
        out[...] = kl[...] / jnp.maximum(kca_r[...] + kcb_r[...], 1.0)

    return pl.pallas_call(
        abody, grid=grid,
        in_specs=[pl.BlockSpec((B, D), lambda i: (i, 0)),
                  pl.BlockSpec((B, 1), lambda i: (i, 0)),
                  pl.BlockSpec((B, 1), lambda i: (i, 0))],
        out_specs=pl.BlockSpec((B, D), lambda i: (i, 0)),
        out_shape=jax.ShapeDtypeStruct((NA, D), F32),
    )(kg_sum, kca, kcb)


def kernel(entity_emb, user_emb, edge_index, edge_type, interact_mat, weight,
           W1, W2):
    head = edge_index[0]
    tail = edge_index[1]
    row = interact_mat[0]
    col = interact_mat[1]
    ent_lo = entity_emb[:, :H]
    ent_hi = entity_emb[:, H:]
    usr_lo = user_emb[:, :H]
    usr_hi = user_emb[:, H:]
    w_lo = jnp.tile(weight[:, :H], (NTILE, 1))
    w_hi = jnp.tile(weight[:, H:], (NTILE, 1))

    npad = E_PAD - EDG
    zpad = jnp.zeros((npad,), I32)
    trash_ent = (jnp.arange(npad, dtype=I32) % (RP_ENT - NE)) + NE
    trash_itm = (jnp.arange(npad, dtype=I32) % (RP_ITM - NI)) + NI

    head_p = _pad_idx(head, trash_ent, 128)
    tail_p = _pad_idx(tail, zpad, 128)
    type_p = _pad_idx(edge_type, zpad, 128)
    rowg_p = _pad_idx(row, zpad, 256)
    row_p = _pad_idx(row, trash_ent, 256)
    colg_p = _pad_idx(col, zpad, 256)
    col_p = _pad_idx(col, trash_itm, 256)

    ones128 = jnp.ones((128,), F32)
    ones256 = jnp.ones((256,), F32)
    z_ent_rows = jnp.zeros((Q_ENT, H), F32)
    z_ent_cnt = jnp.zeros((RP_ENT,), F32)
    z_itm_rows = jnp.zeros((Q_ITM, H), F32)
    z_itm_cnt = jnp.zeros((RP_ITM,), F32)

    kg_k = _make_sc_kernel(RP_ENT, Q_ENT, 128, 20, 128, True, True)
    kg_sum, kg_ca, kg_cb = kg_k(ent_lo, ent_hi, tail_p, head_p, type_p,
                                w_lo, w_hi, z_ent_rows, z_ent_cnt, ones128)

    int_k = _make_sc_kernel(RP_ITM, Q_ITM, 512, 20, 256, False, True)
    int_sum, int_ca, int_cb = int_k(usr_lo, usr_hi, rowg_p, col_p,
                                    z_itm_rows, z_itm_cnt, ones256)

    fus, fus_lo, fus_hi, kg_cat, int_cat = _fusion_call(
        kg_sum[:NI], kg_ca[:NI, None], kg_cb[:NI, None],
        int_sum[:NI], int_ca[:NI, None], int_cb[:NI, None], W1, W2)

    usr_k = _make_sc_kernel(RP_ENT, Q_ENT, 256, 20, 256, False, False)
    user_agg_p = usr_k(fus_lo, fus_hi, colg_p, row_p, z_ent_rows)
    if isinstance(user_agg_p, (tuple, list)):
        user_agg_p = user_agg_p[0]

    att = _att_div_call(kg_sum[NI:NE], kg_ca[NI:NE, None], kg_cb[NI:NE, None])
    final_entity = jnp.concatenate([fus, att], axis=0)
    return final_entity, user_agg_p[:NU], kg_cat, int_cat
